# Initial kernel scaffold; baseline (speedup 1.0000x reference)
import jax
import jax.numpy as jnp
from jax.experimental import pallas as pl


def kernel(batch_size, family_ids, mu, log_sigma):
    raise NotImplementedError("write your pallas kernel here")


# speedup vs baseline: 1.3345x; 1.3345x over previous
import functools

import jax
import jax.numpy as jnp
from jax import lax
from jax.experimental import pallas as pl
from jax.experimental.pallas import tpu as pltpu
from jax.experimental.pallas import tpu_sc as plsc

_NC = 2
_NS = 16
_NW = _NC * _NS
_L = 16


def _sc_sample(ids, mu, log_sigma, eps):
    B = ids.shape[0]
    b_per_w = B // _NW
    mesh = plsc.VectorSubcoreMesh(core_axis_name="c", subcore_axis_name="s")

    @functools.partial(
        pl.kernel,
        mesh=mesh,
        out_type=jax.ShapeDtypeStruct((B,), jnp.float32),
        scratch_types=[
            pltpu.VMEM((b_per_w,), jnp.int32),
            pltpu.VMEM((b_per_w,), jnp.float32),
            pltpu.VMEM((b_per_w,), jnp.float32),
            pltpu.VMEM((b_per_w,), jnp.float32),
            pltpu.SemaphoreType.DMA,
        ],
    )
    def k(ids_hbm, mu_hbm, ls_hbm, eps_hbm, out_hbm, idx_v, mu_v, ls_v, eps_v, sem):
        wid = lax.axis_index("s") * _NC + lax.axis_index("c")
        base = wid * b_per_w
        pltpu.sync_copy(ids_hbm.at[pl.ds(base, b_per_w)], idx_v)
        g1 = pltpu.async_copy(mu_hbm.at[idx_v], mu_v, sem)
        g2 = pltpu.async_copy(ls_hbm.at[idx_v], ls_v, sem)
        pltpu.sync_copy(eps_hbm.at[pl.ds(base, b_per_w)], eps_v)
        g1.wait()
        g2.wait()

        def body(i, carry):
            s = pl.ds(i * _L, _L)
            c = mu_v[s] + jnp.exp(ls_v[s]) * eps_v[s]
            mu_v[s] = jnp.maximum(c, jnp.float32(1e-6))
            return carry

        lax.fori_loop(0, b_per_w // _L, body, 0, unroll=4)
        pltpu.sync_copy(mu_v, out_hbm.at[pl.ds(base, b_per_w)])

    return k(ids, mu, log_sigma, eps)


def kernel(batch_size, family_ids, mu, log_sigma):
    ids = family_ids.astype(jnp.int32)
    B = ids.shape[0]
    eps = jax.random.normal(jax.random.key(42), (B,), dtype=jnp.float32)
    return _sc_sample(ids, mu, log_sigma, eps)
